# baseline (speedup 1.0000x reference)
import functools

import jax
import jax.numpy as jnp
from jax import lax
from jax.experimental import pallas as pl
from jax.experimental.pallas import tpu as pltpu
from jax.experimental.pallas import tpu_sc as plsc

D_EMBED = 64
EMB_SCALE = 8.0
NUM_WORKERS = 32
GATHER_BLK = 128
LANES = 16


def _sc_embed(idx, emb_table, steps):
    mesh = plsc.VectorSubcoreMesh(core_axis_name="c", subcore_axis_name="s")

    @functools.partial(
        pl.kernel,
        mesh=mesh,
        out_type=jax.ShapeDtypeStruct(
            (NUM_WORKERS, steps, GATHER_BLK, D_EMBED), jnp.float32
        ),
        scratch_types=[
            pltpu.VMEM((steps, GATHER_BLK), jnp.int32),
            pltpu.VMEM((GATHER_BLK, D_EMBED), jnp.float32),
            pltpu.SemaphoreType.DMA,
        ],
        compiler_params=pltpu.CompilerParams(use_tc_tiling_on_sc=False),
    )
    def k(idx_hbm, table_hbm, out_hbm, idx_v, rows_v, sem):
        wid = lax.axis_index("s") * 2 + lax.axis_index("c")
        pltpu.sync_copy(idx_hbm.at[wid], idx_v)

        def step(j, carry):
            pltpu.async_copy(table_hbm.at[idx_v.at[j]], rows_v, sem).wait()

            def scale_row(r, c2):
                for c in range(D_EMBED // LANES):
                    s = pl.ds(c * LANES, LANES)
                    rows_v[r, s] = rows_v[r, s] * EMB_SCALE
                return c2

            lax.fori_loop(0, GATHER_BLK, scale_row, 0)
            pltpu.sync_copy(rows_v, out_hbm.at[wid, j])
            return carry

        lax.fori_loop(0, steps, step, 0)

    return k(idx, emb_table)


def kernel(inp, emb_table):
    batch, hist = inp.shape
    total = batch * hist
    per_w = total // NUM_WORKERS
    steps = per_w // GATHER_BLK
    idx = inp.reshape(NUM_WORKERS, steps, GATHER_BLK)
    out = _sc_embed(idx, emb_table, steps)
    return out.reshape(batch, hist, D_EMBED)


# speedup vs baseline: 1.2135x; 1.2135x over previous
import functools

import jax
import jax.numpy as jnp
from jax import lax
from jax.experimental import pallas as pl
from jax.experimental.pallas import tpu as pltpu
from jax.experimental.pallas import tpu_sc as plsc

D_EMBED = 64
EMB_SCALE = 8.0
NUM_WORKERS = 32
GATHER_BLK = 128
NBUF = 4
LANES = 16
ROW_UNROLL = 8


def _sc_embed(idx, emb_table, steps):
    mesh = plsc.VectorSubcoreMesh(core_axis_name="c", subcore_axis_name="s")
    rounds = steps // NBUF

    row_buf = pltpu.VMEM((GATHER_BLK, D_EMBED), jnp.float32)
    scratch = (
        [pltpu.VMEM((steps, GATHER_BLK), jnp.int32)]
        + [row_buf] * NBUF
        + [row_buf] * NBUF
        + [pltpu.SemaphoreType.DMA] * (2 * NBUF)
    )

    @functools.partial(
        pl.kernel,
        mesh=mesh,
        out_type=jax.ShapeDtypeStruct(
            (NUM_WORKERS, steps, GATHER_BLK, D_EMBED), jnp.float32
        ),
        scratch_types=scratch,
        compiler_params=pltpu.CompilerParams(use_tc_tiling_on_sc=False),
    )
    def k(idx_hbm, table_hbm, out_hbm, idx_v, *bufs):
        gbufs = bufs[:NBUF]
        sbufs = bufs[NBUF : 2 * NBUF]
        gsems = bufs[2 * NBUF : 3 * NBUF]
        ssems = bufs[3 * NBUF :]

        wid = lax.axis_index("s") * 2 + lax.axis_index("c")
        pltpu.sync_copy(idx_hbm.at[wid], idx_v)

        def gather(j, b):
            pltpu.async_copy(table_hbm.at[idx_v.at[j]], gbufs[b], gsems[b])

        def gather_wait(j, b):
            pltpu.make_async_copy(
                table_hbm.at[idx_v.at[j]], gbufs[b], gsems[b]
            ).wait()

        def store(j, b):
            pltpu.async_copy(sbufs[b], out_hbm.at[wid, j], ssems[b])

        def store_wait(j, b):
            pltpu.make_async_copy(sbufs[b], out_hbm.at[wid, j], ssems[b]).wait()

        def scale(b):
            g, s = gbufs[b], sbufs[b]

            def body(i, carry):
                r0 = i * ROW_UNROLL
                for rr in range(ROW_UNROLL):
                    for c in range(D_EMBED // LANES):
                        sl = pl.ds(c * LANES, LANES)
                        s[r0 + rr, sl] = g[r0 + rr, sl] * EMB_SCALE
                return carry

            lax.fori_loop(0, GATHER_BLK // ROW_UNROLL, body, 0)

        for b in range(NBUF):
            gather(b, b)

        def visit(rnd, carry):
            for b in range(NBUF):
                j = rnd * NBUF + b
                gather_wait(j, b)

                @pl.when(rnd > 0)
                def _():
                    store_wait(j - NBUF, b)

                scale(b)
                store(j, b)

                @pl.when(rnd < rounds - 1)
                def _():
                    gather(j + NBUF, b)

            return carry

        lax.fori_loop(0, rounds, visit, 0)

        for b in range(NBUF):
            store_wait(steps - NBUF + b, b)

    return k(idx, emb_table)


def kernel(inp, emb_table):
    batch, hist = inp.shape
    total = batch * hist
    per_w = total // NUM_WORKERS
    steps = per_w // GATHER_BLK
    idx = inp.reshape(NUM_WORKERS, steps, GATHER_BLK)
    out = _sc_embed(idx, emb_table, steps)
    return out.reshape(batch, hist, D_EMBED)
